# Initial kernel scaffold; baseline (speedup 1.0000x reference)
import jax
import jax.numpy as jnp
from jax.experimental import pallas as pl


def kernel(x, edge_index, edge_attr, nan_mask, Wq, bq, Wk, bk, Wv, bv, We, be):
    raise NotImplementedError("write your pallas kernel here")


# speedup vs baseline: 11.1234x; 11.1234x over previous
import functools

import jax
import jax.numpy as jnp
from jax import lax
from jax.experimental import pallas as pl
from jax.experimental.pallas import tpu as pltpu
from jax.experimental.pallas import tpu_sc as plsc

N = 10000
E = 320000
IN_FEATS = 128
EDGE_FEATS = 16
H = 4
D = 32
HD = H * D

NC = 2
NS = 16
L = 16
NW = NC * NS
C = 48
E_PAD = ((E + NW * C - 1) // (NW * C)) * (NW * C)
EW = E_PAD // NW
NCHUNK = EW // C
NPS = 624
NTL = N - NPS * NS


def _node_body(x_ref, wq_ref, wk_ref, wv_ref, waug_ref,
               bq_ref, bk_ref, bv_ref, kv_ref, qgc_ref):
    xb = x_ref[...]
    k = jnp.dot(xb, wk_ref[...].T, preferred_element_type=jnp.float32) + bk_ref[...]
    v = jnp.dot(xb, wv_ref[...].T, preferred_element_type=jnp.float32) + bv_ref[...]
    kv_ref[...] = jnp.concatenate([k, v], axis=1)
    qs = (jnp.dot(xb, wq_ref[...].T, preferred_element_type=jnp.float32)
          + bq_ref[...]) * jnp.float32(D ** -0.5)
    waug = waug_ref[...]
    cols = [qs]
    for h in range(H):
        gh = jnp.dot(qs[:, h * D:(h + 1) * D], waug[h * D:(h + 1) * D, :],
                     preferred_element_type=jnp.float32)
        cols.append(gh)
        cols.append(jnp.zeros((qs.shape[0], 32 - 17), jnp.float32))
    qgc_ref[...] = jnp.concatenate(cols, axis=1)


def _edge_body(kv_hbm, qgc_hbm, eaa_hbm, src_hbm, dst_hbm, zo_hbm, zd_hbm,
               out_hbm, den_hbm,
               src_v, dst_v, kv_v, qgc_v, ea_v, msg_v, pden_v, pw_v,
               out_s, den_s, sem1, sem2):
    cid = lax.axis_index("c")
    sid = lax.axis_index("s")
    wid = sid * NC + cid

    zero16 = jnp.zeros((L,), jnp.float32)

    pltpu.sync_copy(zo_hbm.at[pl.ds(sid * NPS, NPS)],
                    out_s.at[pl.ds(sid * NPS, NPS)])
    pltpu.sync_copy(zd_hbm.at[pl.ds(sid * NPS, NPS)],
                    den_s.at[pl.ds(sid * NPS, NPS)])

    @pl.when(sid == NS - 1)
    def _zero_tail():
        pltpu.sync_copy(zo_hbm.at[pl.ds(NPS * NS, NTL)],
                        out_s.at[pl.ds(NPS * NS, NTL)])
        pltpu.sync_copy(zd_hbm.at[pl.ds(NPS * NS, NTL)],
                        den_s.at[pl.ds(NPS * NS, NTL)])

    def _zp(i, _):
        pden_v[i, :] = zero16
        return 0
    lax.fori_loop(0, C, _zp, 0)

    plsc.subcore_barrier()

    lanes = lax.iota(jnp.int32, L)

    def chunk(i, _):
        base = wid * EW + i * C
        pltpu.sync_copy(src_hbm.at[pl.ds(base, C)], src_v)
        pltpu.sync_copy(dst_hbm.at[pl.ds(base, C)], dst_v)
        pltpu.sync_copy(eaa_hbm.at[pl.ds(base, C)], ea_v)
        cp1 = pltpu.async_copy(kv_hbm.at[src_v], kv_v, sem1)
        cp2 = pltpu.async_copy(qgc_hbm.at[dst_v], qgc_v, sem2)
        cp1.wait()
        cp2.wait()

        for g in range(C // L):
            rows = g * L + lanes

            def qk(d, accs):
                out = []
                for h in range(H):
                    col = jnp.full((L,), h * D, jnp.int32) + d
                    kvec = plsc.load_gather(kv_v, [rows, col])
                    qvec = plsc.load_gather(qgc_v, [rows, col])
                    out.append(accs[h] + kvec * qvec)
                return tuple(out)
            accs = lax.fori_loop(0, D, qk, (zero16,) * H)

            def ge(f, accs):
                fcol = jnp.full((L,), 0, jnp.int32) + f
                evec = plsc.load_gather(ea_v, [rows, fcol])
                out = []
                for h in range(H):
                    gcol = jnp.full((L,), HD + h * D, jnp.int32) + f
                    gvec = plsc.load_gather(qgc_v, [rows, gcol])
                    out.append(accs[h] + gvec * evec)
                return tuple(out)
            accs = lax.fori_loop(0, EDGE_FEATS + 1, ge, accs)

            nf = plsc.load_gather(ea_v, [rows, jnp.full((L,), 17, jnp.int32)])
            valid = plsc.load_gather(ea_v, [rows, jnp.full((L,), 18, jnp.int32)])
            isnan = nf > jnp.float32(0.5)
            for h in range(H):
                a = jnp.where(isnan, jnp.float32(1e-9), accs[h])
                a = jnp.minimum(jnp.maximum(a, jnp.float32(-60.0)),
                                jnp.float32(60.0))
                p = jnp.exp(a) * valid
                pw = jnp.where(isnan, jnp.float32(0.0), p)
                plsc.store_scatter(pden_v, [rows, jnp.full((L,), h, jnp.int32)], p)
                plsc.store_scatter(pw_v, [rows * H + h], pw)

        def msg(c, _):
            for h in range(H):
                pwb = plsc.load_gather(pw_v, [jnp.full((L,), c * H + h, jnp.int32)])
                for j in range(2):
                    vvec = kv_v[c, pl.ds(HD + h * D + j * L, L)]
                    msg_v[c, pl.ds(h * D + j * L, L)] = vvec * pwb
            return 0
        lax.fori_loop(0, C, msg, 0)

        pltpu.sync_copy(pden_v, den_s.at[dst_v], add=True)
        pltpu.sync_copy(msg_v, out_s.at[dst_v], add=True)
        return 0

    lax.fori_loop(0, NCHUNK, chunk, 0)
    plsc.subcore_barrier()

    pltpu.sync_copy(out_s.at[pl.ds(sid * NPS, NPS)],
                    out_hbm.at[cid, pl.ds(sid * NPS, NPS)])
    pltpu.sync_copy(den_s.at[pl.ds(sid * NPS, NPS)],
                    den_hbm.at[cid, pl.ds(sid * NPS, NPS)])

    @pl.when(sid == NS - 1)
    def _publish_tail():
        pltpu.sync_copy(out_s.at[pl.ds(NPS * NS, NTL)],
                        out_hbm.at[cid, pl.ds(NPS * NS, NTL)])
        pltpu.sync_copy(den_s.at[pl.ds(NPS * NS, NTL)],
                        den_hbm.at[cid, pl.ds(NPS * NS, NTL)])


def _final_body(op_ref, dp_ref, x_ref, o_ref):
    acc = op_ref[0] + op_ref[1]
    den = dp_ref[0][:, :H] + dp_ref[1][:, :H]
    denb = jnp.broadcast_to(den[:, :, None], (den.shape[0], H, D))
    denb = denb.reshape(den.shape[0], HD)
    safe = denb > jnp.float32(0.0)
    o_ref[...] = jnp.where(safe, acc / denb, jnp.float32(0.0)) + x_ref[...]


def kernel(x, edge_index, edge_attr, nan_mask, Wq, bq, Wk, bk, Wv, bv, We, be):
    BN = 2000
    waug = jnp.concatenate([We, be[:, None]], axis=1)
    kv, qgc = pl.pallas_call(
        _node_body,
        grid=(N // BN,),
        in_specs=[
            pl.BlockSpec((BN, IN_FEATS), lambda i: (i, 0)),
            pl.BlockSpec((HD, IN_FEATS), lambda i: (0, 0)),
            pl.BlockSpec((HD, IN_FEATS), lambda i: (0, 0)),
            pl.BlockSpec((HD, IN_FEATS), lambda i: (0, 0)),
            pl.BlockSpec((HD, 17), lambda i: (0, 0)),
            pl.BlockSpec((1, HD), lambda i: (0, 0)),
            pl.BlockSpec((1, HD), lambda i: (0, 0)),
            pl.BlockSpec((1, HD), lambda i: (0, 0)),
        ],
        out_specs=[
            pl.BlockSpec((BN, 2 * HD), lambda i: (i, 0)),
            pl.BlockSpec((BN, 2 * HD), lambda i: (i, 0)),
        ],
        out_shape=[jax.ShapeDtypeStruct((N, 2 * HD), jnp.float32)] * 2,
    )(x, Wq, Wk, Wv, waug, bq[None, :], bk[None, :], bv[None, :])

    nanf = nan_mask.astype(jnp.float32)
    eaa = jnp.concatenate(
        [edge_attr, jnp.ones((E, 1), jnp.float32), nanf[:, None],
         jnp.ones((E, 1), jnp.float32),
         jnp.zeros((E, 13), jnp.float32)], axis=1)
    eaa = jnp.concatenate([eaa, jnp.zeros((E_PAD - E, 32), jnp.float32)], axis=0)
    srcp = jnp.concatenate([edge_index[0], jnp.zeros((E_PAD - E,), jnp.int32)])
    dstp = jnp.concatenate([edge_index[1], jnp.zeros((E_PAD - E,), jnp.int32)])

    mesh = plsc.VectorSubcoreMesh(core_axis_name="c", subcore_axis_name="s",
                                  num_cores=NC, num_subcores=NS)
    out_p, den_p = pl.kernel(
        _edge_body,
        out_type=[jax.ShapeDtypeStruct((NC, N, HD), jnp.float32),
                  jax.ShapeDtypeStruct((NC, N, L), jnp.float32)],
        mesh=mesh,
        compiler_params=pltpu.CompilerParams(use_tc_tiling_on_sc=False,
                                             needs_layout_passes=False),
        scratch_types=[
            pltpu.VMEM((C,), jnp.int32),
            pltpu.VMEM((C,), jnp.int32),
            pltpu.VMEM((C, 2 * HD), jnp.float32),
            pltpu.VMEM((C, 2 * HD), jnp.float32),
            pltpu.VMEM((C, 32), jnp.float32),
            pltpu.VMEM((C, HD), jnp.float32),
            pltpu.VMEM((C, L), jnp.float32),
            pltpu.VMEM((C * H,), jnp.float32),
            pltpu.VMEM_SHARED((N, HD), jnp.float32),
            pltpu.VMEM_SHARED((N, L), jnp.float32),
            pltpu.SemaphoreType.DMA,
            pltpu.SemaphoreType.DMA,
        ],
    )(kv, qgc, eaa, srcp, dstp,
      jnp.zeros((N, HD), jnp.float32), jnp.zeros((N, L), jnp.float32))

    out = pl.pallas_call(
        _final_body,
        grid=(N // BN,),
        in_specs=[
            pl.BlockSpec((NC, BN, HD), lambda i: (0, i, 0)),
            pl.BlockSpec((NC, BN, L), lambda i: (0, i, 0)),
            pl.BlockSpec((BN, HD), lambda i: (i, 0)),
        ],
        out_specs=pl.BlockSpec((BN, HD), lambda i: (i, 0)),
        out_shape=jax.ShapeDtypeStruct((N, HD), jnp.float32),
    )(out_p, den_p, x)
    return out.reshape(N, H, D)
